# Initial kernel scaffold; baseline (speedup 1.0000x reference)
import jax
import jax.numpy as jnp
from jax.experimental import pallas as pl


def kernel(z, embeddings):
    raise NotImplementedError("write your pallas kernel here")


# speedup vs baseline: 1.0531x; 1.0531x over previous
import functools

import jax
import jax.numpy as jnp
from jax import lax
from jax.experimental import pallas as pl
from jax.experimental.pallas import tpu as pltpu, tpu_sc as plsc

_N_CODES = 8192
_D = 32
_BM = 1024
_BN = 2048
_COMMIT = 0.25


def _argmin_body(z_ref, eT_ref, idx_ref, loss_ref, dbg_ref,
                 dmin_s, amin_s, acc_ref):
    m = pl.program_id(0)
    n = pl.program_id(1)
    n_last = pl.num_programs(1) - 1
    zb = z_ref[...]
    a = jnp.sum(zb * zb, axis=1, keepdims=True)
    z2 = (2.0 * zb).astype(jnp.bfloat16)
    eT = eT_ref[...]
    b = jnp.dot(z2, eT, preferred_element_type=jnp.float32)
    esq = jnp.sum(eT * eT, axis=0, keepdims=True)
    d = (a - b) + esq
    dmin_c = jnp.min(d, axis=1, keepdims=True)
    cols = lax.broadcasted_iota(jnp.int32, (_BM, _BN), 1) + n * _BN
    amin_c = jnp.min(
        jnp.where(d == dmin_c, cols, jnp.int32(2**31 - 1)),
        axis=1, keepdims=True)

    @pl.when(n == 0)
    def _init_row():
        dmin_s[...] = dmin_c
        amin_s[...] = amin_c

    @pl.when(n > 0)
    def _merge():
        bv = dmin_s[...].astype(jnp.bfloat16).astype(jnp.float32)
        better = dmin_c < bv
        dmin_s[...] = jnp.where(better, dmin_c, bv)
        amin_s[...] = jnp.where(better, amin_c, amin_s[...])

    @pl.when(n == n_last)
    def _emit():
        idx_ref[...] = amin_s[...]
        dbg_ref[...] = dmin_s[...]

        @pl.when(m == 0)
        def _init_acc():
            acc_ref[0] = 0.0

        acc_ref[0] += jnp.sum(dmin_s[...])

        @pl.when(m == pl.num_programs(0) - 1)
        def _fin():
            loss_ref[0, 0] = acc_ref[0] * ((1.0 + _COMMIT) / (8 * 1024 * _D))


def _tc_argmin(flat_z, embeddings_t):
    n_rows = flat_z.shape[0]
    return pl.pallas_call(
        _argmin_body,
        grid=(n_rows // _BM, _N_CODES // _BN),
        in_specs=[
            pl.BlockSpec((_BM, _D), lambda m, n: (m, 0)),
            pl.BlockSpec((_D, _BN), lambda m, n: (0, n)),
        ],
        out_specs=[
            pl.BlockSpec((_BM, 1), lambda m, n: (m, 0)),
            pl.BlockSpec(memory_space=pltpu.SMEM),
            pl.BlockSpec((_BM, 1), lambda m, n: (m, 0)),
        ],
        out_shape=[
            jax.ShapeDtypeStruct((n_rows, 1), jnp.int32),
            jax.ShapeDtypeStruct((1, 1), jnp.float32),
            jax.ShapeDtypeStruct((n_rows, 1), jnp.float32),
        ],
        scratch_shapes=[
            pltpu.VMEM((_BM, 1), jnp.float32),
            pltpu.VMEM((_BM, 1), jnp.int32),
            pltpu.SMEM((1,), jnp.float32),
        ],
        compiler_params=pltpu.CompilerParams(
            dimension_semantics=("arbitrary", "arbitrary")),
    )(flat_z, embeddings_t)


def _sc_gather_st(table, idx2d, flat_z):
    n_rows = flat_z.shape[0]
    info = plsc.get_sparse_core_info()
    nw = info.num_cores * info.num_subcores
    bpw = n_rows // nw
    n_chunks = bpw // 128
    mesh = plsc.VectorSubcoreMesh(core_axis_name="c", subcore_axis_name="s")

    @functools.partial(
        pl.kernel, mesh=mesh,
        compiler_params=pltpu.CompilerParams(use_tc_tiling_on_sc=False),
        out_type=jax.ShapeDtypeStruct((n_rows, _D), jnp.float32),
        scratch_types=[
            pltpu.VMEM((n_chunks, 128), jnp.int32),
            pltpu.VMEM((bpw, _D), jnp.float32),
            pltpu.VMEM((bpw, _D), jnp.float32),
            pltpu.SemaphoreType.DMA,
        ],
    )
    def body(table_hbm, idx_hbm, z_hbm, out_hbm, idx_v, rows_v, z_v, sem):
        wid = lax.axis_index("s") * info.num_cores + lax.axis_index("c")
        base = wid * bpw
        pltpu.sync_copy(idx_hbm.at[pl.ds(wid * n_chunks, n_chunks)], idx_v)
        cps = [
            pltpu.async_copy(table_hbm.at[idx_v.at[j]],
                             rows_v.at[pl.ds(j * 128, 128)], sem)
            for j in range(n_chunks)
        ]
        pltpu.sync_copy(z_hbm.at[pl.ds(base, bpw)], z_v)
        for cp in cps:
            cp.wait()

        def row(r, _):
            for h in range(_D // 16):
                q = rows_v[r, pl.ds(h * 16, 16)]
                zz = z_v[r, pl.ds(h * 16, 16)]
                rows_v[r, pl.ds(h * 16, 16)] = zz + (q - zz)
            return 0

        lax.fori_loop(0, bpw, row, 0)
        pltpu.sync_copy(rows_v, out_hbm.at[pl.ds(base, bpw)])

    return body(table, idx2d, flat_z)


def kernel(z, embeddings):
    flat_z = z.reshape(-1, _D)
    idx, loss, _ = _tc_argmin(flat_z, embeddings.T)
    idx2d = idx.reshape(-1, 128)
    qst = _sc_gather_st(embeddings, idx2d, flat_z)
    return qst.reshape(z.shape), loss[0, 0]
